# baseline (speedup 1.0000x reference)
import functools

import jax
import jax.numpy as jnp
from jax import lax
from jax.experimental import pallas as pl
from jax.experimental.pallas import tpu as pltpu

_NEG_BIG = -1e30


def _router_body(x_ref, w1_ref, b1_ref, w2_ref, b2_ref,
                 out_ref, aux_ref,
                 freq_acc, prob_acc, *, num_blocks, num_tokens, chunks):
    i = pl.program_id(0)

    @pl.when(i == 0)
    def _init():
        freq_acc[...] = jnp.zeros_like(freq_acc)
        prob_acc[...] = jnp.zeros_like(prob_acc)

    w1 = w1_ref[...]
    w2 = w2_ref[...]
    b1 = b1_ref[...]
    b2 = b2_ref[...]
    bt = x_ref.shape[0]
    ne = w2.shape[1]
    sz = bt // chunks

    freq_parts, prob_parts = [], []
    for c in range(chunks):
        sl = pl.ds(c * sz, sz)
        x = x_ref[sl, :]
        h = jnp.dot(x, w1, preferred_element_type=jnp.float32)
        h = h + b1
        h = h * (0.5 * jnp.tanh(0.5 * h) + 0.5)
        lt = lax.dot_general(w2, h, (((0,), (1,)), ((), ())),
                             preferred_element_type=jnp.float32)
        lt = lt + b2

        row = lax.broadcasted_iota(jnp.int32, (ne, sz), 0)

        m1 = jnp.max(lt, axis=0, keepdims=True)
        i1 = jnp.min(jnp.where(lt == m1, row, ne), axis=0, keepdims=True)
        masked = jnp.where(row == i1, _NEG_BIG, lt)
        m2 = jnp.max(masked, axis=0, keepdims=True)
        i2 = jnp.min(jnp.where(masked == m2, row, ne), axis=0, keepdims=True)

        e2 = jnp.exp(m2 - m1)
        inv = 1.0 / (1.0 + e2)
        out_ref[:, sl] = jnp.concatenate(
            [inv, e2 * inv, i1.astype(jnp.float32), i2.astype(jnp.float32)],
            axis=0)

        e = jnp.exp(lt - m1)
        probs = e * (1.0 / jnp.sum(e, axis=0, keepdims=True))
        prob_parts.append(jnp.sum(probs, axis=1, keepdims=True))
        freq_parts.append(jnp.sum(
            jnp.where(row == i1, 1.0, 0.0), axis=1, keepdims=True))

    prob_acc[...] += sum(prob_parts)
    freq_acc[...] += sum(freq_parts)

    @pl.when(i == num_blocks - 1)
    def _finish():
        scale = ne / (float(num_tokens) * float(num_tokens))
        aux_ref[...] = (scale * jnp.sum(freq_acc[...] * prob_acc[...],
                                        keepdims=True)).reshape(1, 1)


@functools.partial(jax.jit, static_argnames=())
def kernel(x, W1, b1, W2, b2):
    num_tokens, embed = x.shape
    hidden = W1.shape[1]
    ne = W2.shape[1]
    bt = 4096
    chunks = 4
    num_blocks = num_tokens // bt

    body = functools.partial(_router_body, num_blocks=num_blocks,
                             num_tokens=num_tokens, chunks=chunks)
    packed, aux = pl.pallas_call(
        body,
        grid=(num_blocks,),
        in_specs=[
            pl.BlockSpec((bt, embed), lambda i: (i, 0)),
            pl.BlockSpec((embed, hidden), lambda i: (0, 0)),
            pl.BlockSpec((1, hidden), lambda i: (0, 0)),
            pl.BlockSpec((hidden, ne), lambda i: (0, 0)),
            pl.BlockSpec((ne, 1), lambda i: (0, 0)),
        ],
        out_specs=[
            pl.BlockSpec((4, bt), lambda i: (0, i)),
            pl.BlockSpec((1, 1), lambda i: (0, 0)),
        ],
        out_shape=[
            jax.ShapeDtypeStruct((4, num_tokens), jnp.float32),
            jax.ShapeDtypeStruct((1, 1), jnp.float32),
        ],
        scratch_shapes=[
            pltpu.VMEM((ne, 1), jnp.float32),
            pltpu.VMEM((ne, 1), jnp.float32),
        ],
    )(x, W1, b1.reshape(1, hidden), W2, b2.reshape(ne, 1))
    pt = packed.T
    return pt[:, :2], pt[:, 2:4].astype(jnp.int32), aux.reshape(())


# speedup vs baseline: 1.0059x; 1.0059x over previous
import functools

import jax
import jax.numpy as jnp
from jax import lax
from jax.experimental import pallas as pl
from jax.experimental.pallas import tpu as pltpu

_NEG_BIG = -1e30


def _router_body(x_ref, w1_ref, b1_ref, w2_ref, b2_ref,
                 wts_ref, idx_ref, aux_ref,
                 freq_acc, prob_acc, *, num_blocks, num_tokens, chunks):
    i = pl.program_id(0)

    @pl.when(i == 0)
    def _init():
        freq_acc[...] = jnp.zeros_like(freq_acc)
        prob_acc[...] = jnp.zeros_like(prob_acc)

    w1 = w1_ref[...]
    w2 = w2_ref[...]
    b1 = b1_ref[...]
    b2 = b2_ref[...]
    bt = x_ref.shape[0]
    ne = w2.shape[1]
    sz = bt // chunks

    freq_parts, prob_parts = [], []
    for c in range(chunks):
        sl = pl.ds(c * sz, sz)
        x = x_ref[sl, :]
        h = jnp.dot(x, w1, preferred_element_type=jnp.float32)
        h = h + b1
        h = h * (0.5 * jnp.tanh(0.5 * h) + 0.5)
        lt = lax.dot_general(w2, h, (((0,), (1,)), ((), ())),
                             preferred_element_type=jnp.float32)
        lt = lt + b2

        row = lax.broadcasted_iota(jnp.int32, (ne, sz), 0)

        m1 = jnp.max(lt, axis=0, keepdims=True)
        i1 = jnp.min(jnp.where(lt == m1, row, ne), axis=0, keepdims=True)
        masked = jnp.where(row == i1, _NEG_BIG, lt)
        m2 = jnp.max(masked, axis=0, keepdims=True)
        i2 = jnp.min(jnp.where(masked == m2, row, ne), axis=0, keepdims=True)

        e2 = jnp.exp(m2 - m1)
        inv = 1.0 / (1.0 + e2)
        wts_ref[:, sl] = jnp.concatenate([inv, e2 * inv], axis=0)
        idx_ref[:, sl] = jnp.concatenate([i1, i2], axis=0)

        e = jnp.exp(lt - m1)
        probs = e * (1.0 / jnp.sum(e, axis=0, keepdims=True))
        prob_parts.append(jnp.sum(probs, axis=1, keepdims=True))
        freq_parts.append(jnp.sum(
            jnp.where(row == i1, 1.0, 0.0), axis=1, keepdims=True))

    prob_acc[...] += sum(prob_parts)
    freq_acc[...] += sum(freq_parts)

    @pl.when(i == num_blocks - 1)
    def _finish():
        scale = ne / (float(num_tokens) * float(num_tokens))
        aux_ref[...] = (scale * jnp.sum(freq_acc[...] * prob_acc[...],
                                        keepdims=True)).reshape(1, 1)


@functools.partial(jax.jit, static_argnames=())
def kernel(x, W1, b1, W2, b2):
    num_tokens, embed = x.shape
    hidden = W1.shape[1]
    ne = W2.shape[1]
    bt = 4096
    chunks = 2
    num_blocks = num_tokens // bt

    body = functools.partial(_router_body, num_blocks=num_blocks,
                             num_tokens=num_tokens, chunks=chunks)
    wts_t, idx_t, aux = pl.pallas_call(
        body,
        grid=(num_blocks,),
        in_specs=[
            pl.BlockSpec((bt, embed), lambda i: (i, 0)),
            pl.BlockSpec((embed, hidden), lambda i: (0, 0)),
            pl.BlockSpec((1, hidden), lambda i: (0, 0)),
            pl.BlockSpec((hidden, ne), lambda i: (0, 0)),
            pl.BlockSpec((ne, 1), lambda i: (0, 0)),
        ],
        out_specs=[
            pl.BlockSpec((2, bt), lambda i: (0, i)),
            pl.BlockSpec((2, bt), lambda i: (0, i)),
            pl.BlockSpec((1, 1), lambda i: (0, 0)),
        ],
        out_shape=[
            jax.ShapeDtypeStruct((2, num_tokens), jnp.float32),
            jax.ShapeDtypeStruct((2, num_tokens), jnp.int32),
            jax.ShapeDtypeStruct((1, 1), jnp.float32),
        ],
        scratch_shapes=[
            pltpu.VMEM((ne, 1), jnp.float32),
            pltpu.VMEM((ne, 1), jnp.float32),
        ],
    )(x, W1, b1.reshape(1, hidden), W2, b2.reshape(ne, 1))
    return wts_t.T, idx_t.T, aux.reshape(())


# speedup vs baseline: 1.0488x; 1.0427x over previous
import functools

import jax
import jax.numpy as jnp
from jax import lax
from jax.experimental import pallas as pl
from jax.experimental.pallas import tpu as pltpu

_NEG_BIG = -1e30


def _router_body(x_ref, w1_ref, b1_ref, w2_ref, b2_ref,
                 wts_ref, idx_ref, aux_ref,
                 freq_acc, prob_acc, *, num_blocks, num_tokens, chunks):
    i = pl.program_id(0)

    @pl.when(i == 0)
    def _init():
        freq_acc[...] = jnp.zeros_like(freq_acc)
        prob_acc[...] = jnp.zeros_like(prob_acc)

    w1 = w1_ref[...]
    w2 = w2_ref[...]
    b1 = b1_ref[...]
    b2 = b2_ref[...]
    bt = x_ref.shape[0]
    ne = w2.shape[1]
    sz = bt // chunks

    freq_parts, prob_parts = [], []
    for c in range(chunks):
        sl = pl.ds(c * sz, sz)
        x = x_ref[sl, :]
        h = jnp.dot(x, w1, preferred_element_type=jnp.float32)
        h = h + b1
        h = h * (0.5 * jnp.tanh(0.5 * h) + 0.5)
        lt = lax.dot_general(w2, h, (((0,), (1,)), ((), ())),
                             preferred_element_type=jnp.float32)
        lt = lt + b2

        row = lax.broadcasted_iota(jnp.int32, (ne, sz), 0)

        m1 = jnp.max(lt, axis=0, keepdims=True)
        i1 = jnp.min(jnp.where(lt == m1, row, ne), axis=0, keepdims=True)
        masked = jnp.where(row == i1, _NEG_BIG, lt)
        m2 = jnp.max(masked, axis=0, keepdims=True)
        i2 = jnp.min(jnp.where(masked == m2, row, ne), axis=0, keepdims=True)

        e2 = jnp.exp(m2 - m1)
        inv = 1.0 / (1.0 + e2)
        wts_ref[:, sl] = jnp.concatenate([inv, e2 * inv], axis=0)
        idx_ref[:, sl] = jnp.concatenate([i1, i2], axis=0)

        e = jnp.exp(lt - m1)
        probs = e * (1.0 / jnp.sum(e, axis=0, keepdims=True))
        prob_parts.append(jnp.sum(probs, axis=1, keepdims=True))
        freq_parts.append(jnp.sum(
            jnp.where(row == i1, 1.0, 0.0), axis=1, keepdims=True))

    prob_acc[...] += sum(prob_parts)
    freq_acc[...] += sum(freq_parts)

    @pl.when(i == num_blocks - 1)
    def _finish():
        scale = ne / (float(num_tokens) * float(num_tokens))
        aux_ref[...] = (scale * jnp.sum(freq_acc[...] * prob_acc[...],
                                        keepdims=True)).reshape(1, 1)


@functools.partial(jax.jit, static_argnames=())
def kernel(x, W1, b1, W2, b2):
    num_tokens, embed = x.shape
    hidden = W1.shape[1]
    ne = W2.shape[1]
    bt = 4096
    chunks = 4
    num_blocks = num_tokens // bt

    body = functools.partial(_router_body, num_blocks=num_blocks,
                             num_tokens=num_tokens, chunks=chunks)
    wts_t, idx_t, aux = pl.pallas_call(
        body,
        grid=(num_blocks,),
        in_specs=[
            pl.BlockSpec((bt, embed), lambda i: (i, 0)),
            pl.BlockSpec((embed, hidden), lambda i: (0, 0)),
            pl.BlockSpec((1, hidden), lambda i: (0, 0)),
            pl.BlockSpec((hidden, ne), lambda i: (0, 0)),
            pl.BlockSpec((ne, 1), lambda i: (0, 0)),
        ],
        out_specs=[
            pl.BlockSpec((2, bt), lambda i: (0, i)),
            pl.BlockSpec((2, bt), lambda i: (0, i)),
            pl.BlockSpec((1, 1), lambda i: (0, 0)),
        ],
        out_shape=[
            jax.ShapeDtypeStruct((2, num_tokens), jnp.float32),
            jax.ShapeDtypeStruct((2, num_tokens), jnp.int32),
            jax.ShapeDtypeStruct((1, 1), jnp.float32),
        ],
        scratch_shapes=[
            pltpu.VMEM((ne, 1), jnp.float32),
            pltpu.VMEM((ne, 1), jnp.float32),
        ],
    )(x, W1, b1.reshape(1, hidden), W2, b2.reshape(ne, 1))
    return wts_t.T, idx_t.T, aux.reshape(())
